# baseline (speedup 1.0000x reference)
import functools

import jax
import jax.numpy as jnp
from jax import lax
from jax.experimental import pallas as pl
from jax.experimental.pallas import tpu as pltpu
from jax.experimental.pallas import tpu_sc as plsc

_NC = 2
_NS = 16
_NW = _NC * _NS
_L = 16
_CHUNK = 128


def _sc_degree(dst3, ew3, n_pad):
    ch = dst3.shape[1]
    slc = n_pad // _NS
    slcz = -(-slc // _L) * _L
    mesh = plsc.VectorSubcoreMesh(core_axis_name="c", subcore_axis_name="s")

    @functools.partial(
        pl.kernel,
        out_type=jax.ShapeDtypeStruct((_NC, n_pad), jnp.float32),
        mesh=mesh,
        scratch_types=[
            pltpu.VMEM((ch, _CHUNK), jnp.int32),
            pltpu.VMEM((ch, _CHUNK), jnp.float32),
            pltpu.VMEM((slcz,), jnp.float32),
            pltpu.VMEM_SHARED((n_pad,), jnp.float32),
        ],
    )
    def deg_kernel(dst_hbm, ew_hbm, out_hbm, idx_v, val_v, zbuf_v, deg_sh):
        c = lax.axis_index("c")
        s = lax.axis_index("s")
        wid = s * _NC + c

        def zb(i, carry):
            zbuf_v[pl.ds(i * _L, _L)] = jnp.zeros((_L,), jnp.float32)
            return carry

        lax.fori_loop(0, slcz // _L, zb, 0)
        pltpu.sync_copy(zbuf_v.at[pl.ds(0, slc)], deg_sh.at[pl.ds(s * slc, slc)])
        plsc.subcore_barrier()

        pltpu.sync_copy(dst_hbm.at[wid], idx_v)
        pltpu.sync_copy(ew_hbm.at[wid], val_v)

        def body(j, carry):
            pltpu.sync_copy(val_v.at[j], deg_sh.at[idx_v.at[j]], add=True)
            return carry

        lax.fori_loop(0, ch, body, 0)
        plsc.subcore_barrier()
        pltpu.sync_copy(deg_sh.at[pl.ds(s * slc, slc)],
                        out_hbm.at[c, pl.ds(s * slc, slc)])

    return deg_kernel(dst3, ew3)


_RB = 3
_IB = 2
_UNROLL = 6


def _sc_messages(hs, src4, dst4, ew4, n_pad):
    npass, nw, cpp1, _ = src4.shape
    cpp = cpp1 - 1
    h = hs.shape[1]
    nvr = h // _L
    slc = n_pad // _NS
    mesh = plsc.VectorSubcoreMesh(core_axis_name="c", subcore_axis_name="s")

    @functools.partial(
        pl.kernel,
        out_type=jax.ShapeDtypeStruct((_NC, n_pad, h), jnp.float32),
        mesh=mesh,
        scratch_types=[
            [pltpu.VMEM((_CHUNK, h), jnp.float32) for _ in range(2)],
            pltpu.VMEM((cpp1, _CHUNK), jnp.int32),
            pltpu.VMEM((cpp1, _CHUNK), jnp.int32),
            pltpu.VMEM((cpp1, _CHUNK), jnp.float32),
            pltpu.VMEM_SHARED((n_pad, h), jnp.float32),
            [pltpu.SemaphoreType.DMA for _ in range(2)],
        ],
    )
    def msg_kernel(hs_hbm, src_hbm, dst_hbm, ew_hbm, out_hbm,
                   rows, src_v, dst_v, ew_v, acc_sh, gsems):
        c = lax.axis_index("c")
        s = lax.axis_index("s")
        wid = s * _NC + c

        def zr(i, carry):
            def zc(k, carry2):
                rows[0][i, pl.ds(k * _L, _L)] = jnp.zeros((_L,), jnp.float32)
                return carry2
            lax.fori_loop(0, nvr, zc, 0)
            return carry

        lax.fori_loop(0, _CHUNK, zr, 0)
        for t in range(slc // _CHUNK):
            pltpu.sync_copy(rows[0],
                            acc_sh.at[pl.ds(s * slc + t * _CHUNK, _CHUNK)])
        plsc.subcore_barrier()

        def gather(j, b):
            pltpu.async_copy(hs_hbm.at[src_v.at[j]], rows[b], gsems[b])

        def gather_wait(b):
            pltpu.make_async_copy(hs_hbm.at[src_v.at[0]], rows[b],
                                  gsems[b]).wait()

        def scale(b, j):
            buf = rows[b]

            def srow(i16, carry):
                ewv = ew_v[j, pl.ds(i16 * _L, _L)]
                for l in range(_L):
                    wv = jnp.full((_L,), ewv[l], dtype=jnp.float32)
                    row = i16 * _L + l
                    for k in range(nvr):
                        buf[row, pl.ds(k * _L, _L)] = (
                            buf[row, pl.ds(k * _L, _L)] * wv)
                return carry

            lax.fori_loop(0, _CHUNK // _L, srow, 0)

        for p in range(npass):
            pltpu.sync_copy(src_hbm.at[p, wid], src_v)
            pltpu.sync_copy(dst_hbm.at[p, wid], dst_v)
            pltpu.sync_copy(ew_hbm.at[p, wid], ew_v)
            gather(0, 0)

            def outer(g, carry):
                for u in range(2):
                    j = g * 2 + u
                    gather(j + 1, 1 - u)
                    gather_wait(u)
                    scale(u, j)
                    pltpu.sync_copy(rows[u], acc_sh.at[dst_v.at[j]],
                                    add=True)
                return carry

            lax.fori_loop(0, cpp // 2, outer, 0)
            gather_wait(cpp % 2)

        plsc.subcore_barrier()
        for t in range(slc // _CHUNK):
            off = s * slc + t * _CHUNK
            pltpu.sync_copy(acc_sh.at[pl.ds(off, _CHUNK)],
                            out_hbm.at[c, pl.ds(off, _CHUNK)])

    return msg_kernel(hs, src4, dst4, ew4)


def _tc_matmul_scale(x, w, deg2):
    n, f = x.shape
    h = w.shape[1]
    blk = 1000
    deg3 = deg2.reshape(_NC, -1, 1)

    def body(x_ref, w_ref, deg_ref, o_ref):
        d = deg_ref[0] + deg_ref[1] + 1.0
        dinv = lax.rsqrt(d)
        hmat = jnp.dot(x_ref[...], w_ref[...], preferred_element_type=jnp.float32)
        o_ref[...] = hmat * dinv

    return pl.pallas_call(
        body,
        grid=(n // blk,),
        in_specs=[
            pl.BlockSpec((blk, f), lambda i: (i, 0)),
            pl.BlockSpec((f, h), lambda i: (0, 0)),
            pl.BlockSpec((_NC, blk, 1), lambda i: (0, i, 0)),
        ],
        out_specs=pl.BlockSpec((blk, h), lambda i: (i, 0)),
        out_shape=jax.ShapeDtypeStruct((n, h), jnp.float32),
    )(x, w, deg3)


def _tc_pool_mlp(acc, hs, deg2, batch, rho, b1, M1, mb1, M2, mb2, M3, mb3):
    n, h = hs.shape
    g = rho.shape[0]
    out_dim = M3.shape[1]
    blk = 1000
    grid = n // blk
    deg3 = deg2.reshape(_NC, -1, 1)
    bat2 = batch.reshape(n, 1).astype(jnp.int32)
    rho2 = rho.reshape(g, 1)
    b1r = b1.reshape(1, h)
    mb1r = mb1.reshape(1, h)
    mb2r = mb2.reshape(1, h)
    mb3r = mb3.reshape(1, out_dim)
    m1a = M1[:h]
    m1b = M1[h:2 * h]
    m1c = M1[2 * h:]

    def body(acc_ref, hs_ref, deg_ref, bat_ref, rho_ref, b1_ref,
             m1a_ref, m1b_ref, m1c_ref, mb1_ref, m2_ref, mb2_ref,
             m3_ref, mb3_ref, o_ref, hsum_s, hmax_s, cnt_s):
        i = pl.program_id(0)

        @pl.when(i == 0)
        def _():
            hsum_s[...] = jnp.zeros_like(hsum_s)
            hmax_s[...] = jnp.zeros_like(hmax_s)
            cnt_s[...] = jnp.zeros_like(cnt_s)

        d = deg_ref[0] + deg_ref[1] + 1.0
        dinv = lax.rsqrt(d)
        hact = jnp.maximum(
            (acc_ref[0] + acc_ref[1] + hs_ref[...]) * dinv + b1_ref[...], 0.0)
        bat = bat_ref[...]
        gid = lax.broadcasted_iota(jnp.int32, (blk, g), 1)
        mask = (bat == gid).astype(jnp.float32)
        dn = (((0,), (0,)), ((), ()))
        hsum_s[...] += lax.dot_general(mask, hact, dn,
                                       preferred_element_type=jnp.float32)
        cnt_s[...] += lax.dot_general(mask, jnp.ones_like(hact), dn,
                                      preferred_element_type=jnp.float32)

        def gbody(gg, carry):
            m = (bat == gg).astype(jnp.float32)
            colmax = jnp.max(hact * m, axis=0, keepdims=True)
            hmax_s[pl.ds(gg, 1), :] = jnp.maximum(hmax_s[pl.ds(gg, 1), :], colmax)
            return carry

        lax.fori_loop(0, g, gbody, 0)

        @pl.when(i == grid - 1)
        def _():
            hmean = hsum_s[...] / jnp.maximum(cnt_s[...], 1.0)
            hg = (jnp.dot(hmax_s[...], m1a_ref[...],
                          preferred_element_type=jnp.float32)
                  + jnp.dot(hmean, m1b_ref[...],
                            preferred_element_type=jnp.float32)
                  + rho_ref[...] * m1c_ref[...] + mb1_ref[...])
            z = jnp.maximum(hg, 0.0)
            z = jnp.maximum(
                jnp.dot(z, m2_ref[...], preferred_element_type=jnp.float32)
                + mb2_ref[...], 0.0)
            o_ref[...] = (jnp.dot(z, m3_ref[...],
                                  preferred_element_type=jnp.float32)
                          + mb3_ref[...])

    full = lambda shape: pl.BlockSpec(shape, lambda i: tuple(0 for _ in shape))
    return pl.pallas_call(
        body,
        grid=(grid,),
        in_specs=[
            pl.BlockSpec((_NC, blk, h), lambda i: (0, i, 0)),
            pl.BlockSpec((blk, h), lambda i: (i, 0)),
            pl.BlockSpec((_NC, blk, 1), lambda i: (0, i, 0)),
            pl.BlockSpec((blk, 1), lambda i: (i, 0)),
            full((g, 1)),
            full((1, h)),
            full((h, h)),
            full((h, h)),
            full((1, h)),
            full((1, h)),
            full((h, h)),
            full((1, h)),
            full((h, out_dim)),
            full((1, out_dim)),
        ],
        out_specs=pl.BlockSpec((g, out_dim), lambda i: (0, 0)),
        out_shape=jax.ShapeDtypeStruct((g, out_dim), jnp.float32),
        scratch_shapes=[
            pltpu.VMEM((g, h), jnp.float32),
            pltpu.VMEM((g, h), jnp.float32),
            pltpu.VMEM((g, h), jnp.float32),
        ],
    )(acc, hs, deg3, bat2, rho2, b1r, m1a, m1b, m1c, mb1r, M2, mb2r, M3, mb3r)


def kernel(x, edge_index, edge_attr, batch, rho, W1, b1, M1, mb1, M2, mb2, M3, mb3):
    n, _ = x.shape
    e = edge_index.shape[1]

    n_pad = -(-n // 2048) * 2048

    npass = 3
    grp = _NW * _CHUNK
    ch_total = -(-e // grp)
    ch_total = -(-ch_total // (2 * npass)) * (2 * npass)
    e_pad = ch_total * grp
    pad = e_pad - e

    src = edge_index[0].astype(jnp.int32)
    dst = edge_index[1].astype(jnp.int32)
    ew = edge_attr.reshape(-1).astype(jnp.float32)
    if pad:
        fill = (jnp.arange(pad, dtype=jnp.int32) * 1009) % n
        src = jnp.concatenate([src, fill])
        dst = jnp.concatenate([dst, fill])
        ew = jnp.concatenate([ew, jnp.zeros((pad,), jnp.float32)])

    ch = e_pad // grp
    cpp = ch // npass
    src3 = src.reshape(_NW, ch, _CHUNK)
    dst3 = dst.reshape(_NW, ch, _CHUNK)
    ew3 = ew.reshape(_NW, ch, _CHUNK)
    zpad_i = jnp.zeros((npass, _NW, 1, _CHUNK), jnp.int32)
    zpad_f = jnp.zeros((npass, _NW, 1, _CHUNK), jnp.float32)
    src4 = jnp.concatenate(
        [src.reshape(npass, _NW, cpp, _CHUNK), zpad_i], 2)
    dst4 = jnp.concatenate(
        [dst.reshape(npass, _NW, cpp, _CHUNK), zpad_i], 2)
    ew4 = jnp.concatenate(
        [ew.reshape(npass, _NW, cpp, _CHUNK), zpad_f], 2)

    deg2 = _sc_degree(dst3, ew3, n_pad)
    hs = _tc_matmul_scale(x, W1, deg2)
    acc = _sc_messages(hs, src4, dst4, ew4, n_pad)
    return _tc_pool_mlp(acc, hs, deg2, batch, rho,
                        b1, M1, mb1, M2, mb2, M3, mb3)


# speedup vs baseline: 2.3972x; 2.3972x over previous
import functools

import jax
import jax.numpy as jnp
from jax import lax
from jax.experimental import pallas as pl
from jax.experimental.pallas import tpu as pltpu
from jax.experimental.pallas import tpu_sc as plsc

_NC = 2
_NS = 16
_NW = _NC * _NS
_L = 16
_CHUNK = 128


def _sc_degree(dst3, ew3, n_pad):
    ch = dst3.shape[1]
    slc = n_pad // _NS
    slcz = -(-slc // _L) * _L
    mesh = plsc.VectorSubcoreMesh(core_axis_name="c", subcore_axis_name="s")

    @functools.partial(
        pl.kernel,
        out_type=jax.ShapeDtypeStruct((_NC, n_pad), jnp.float32),
        mesh=mesh,
        scratch_types=[
            pltpu.VMEM((ch, _CHUNK), jnp.int32),
            pltpu.VMEM((ch, _CHUNK), jnp.float32),
            pltpu.VMEM((slcz,), jnp.float32),
            pltpu.VMEM_SHARED((n_pad,), jnp.float32),
        ],
    )
    def deg_kernel(dst_hbm, ew_hbm, out_hbm, idx_v, val_v, zbuf_v, deg_sh):
        c = lax.axis_index("c")
        s = lax.axis_index("s")
        wid = s * _NC + c

        def zb(i, carry):
            zbuf_v[pl.ds(i * _L, _L)] = jnp.zeros((_L,), jnp.float32)
            return carry

        lax.fori_loop(0, slcz // _L, zb, 0)
        pltpu.sync_copy(zbuf_v.at[pl.ds(0, slc)], deg_sh.at[pl.ds(s * slc, slc)])
        plsc.subcore_barrier()

        pltpu.sync_copy(dst_hbm.at[wid], idx_v)
        pltpu.sync_copy(ew_hbm.at[wid], val_v)

        def body(j, carry):
            pltpu.sync_copy(val_v.at[j], deg_sh.at[idx_v.at[j]], add=True)
            return carry

        lax.fori_loop(0, ch, body, 0)
        plsc.subcore_barrier()
        pltpu.sync_copy(deg_sh.at[pl.ds(s * slc, slc)],
                        out_hbm.at[c, pl.ds(s * slc, slc)])

    return deg_kernel(dst3, ew3)


_RB = 3
_IB = 2
_UNROLL = 6


def _sc_messages(hs, src3, dst3, ew3, n_pad):
    ch = src3.shape[1]
    h = hs.shape[1]
    nvr = h // _L
    slc = n_pad // _NS
    mesh = plsc.VectorSubcoreMesh(core_axis_name="c", subcore_axis_name="s")

    @functools.partial(
        pl.kernel,
        out_type=jax.ShapeDtypeStruct((_NC, n_pad, h), jnp.float32),
        mesh=mesh,
        scratch_types=[
            pltpu.VMEM((ch, _CHUNK), jnp.int32),
            pltpu.VMEM((ch, _CHUNK), jnp.int32),
            pltpu.VMEM((ch, _CHUNK), jnp.float32),
            pltpu.VMEM((_CHUNK, h), jnp.float32),
            pltpu.VMEM_SHARED((n_pad, h), jnp.float32),
            pltpu.SemaphoreType.DMA,
        ],
    )
    def msg_kernel(hs_hbm, src_hbm, dst_hbm, ew_hbm, out_hbm,
                   src_v, dst_v, ew_v, rows_v, acc_sh, sem):
        c = lax.axis_index("c")
        s = lax.axis_index("s")
        wid = s * _NC + c

        def zr(i, carry):
            def zc(k, carry2):
                rows_v[i, pl.ds(k * _L, _L)] = jnp.zeros((_L,), jnp.float32)
                return carry2
            lax.fori_loop(0, nvr, zc, 0)
            return carry

        lax.fori_loop(0, _CHUNK, zr, 0)

        def zslab(t, carry):
            pltpu.sync_copy(rows_v, acc_sh.at[pl.ds(s * slc + t * _CHUNK, _CHUNK)])
            return carry

        lax.fori_loop(0, slc // _CHUNK, zslab, 0)
        plsc.subcore_barrier()

        pltpu.sync_copy(src_hbm.at[wid], src_v)
        pltpu.sync_copy(dst_hbm.at[wid], dst_v)
        pltpu.sync_copy(ew_hbm.at[wid], ew_v)

        def body(j, carry):
            pltpu.async_copy(hs_hbm.at[src_v.at[j]], rows_v, sem).wait()

            def srow(i16, carry2):
                ewv = ew_v[j, pl.ds(i16 * _L, _L)]
                for l in range(_L):
                    wv = jnp.full((_L,), ewv[l], dtype=jnp.float32)
                    row = i16 * _L + l
                    for k in range(nvr):
                        rows_v[row, pl.ds(k * _L, _L)] = (
                            rows_v[row, pl.ds(k * _L, _L)] * wv)
                return carry2

            lax.fori_loop(0, _CHUNK // _L, srow, 0)
            pltpu.sync_copy(rows_v, acc_sh.at[dst_v.at[j]], add=True)
            return carry

        lax.fori_loop(0, ch, body, 0)
        plsc.subcore_barrier()

        def wr(t, carry):
            off = s * slc + t * _CHUNK
            pltpu.sync_copy(acc_sh.at[pl.ds(off, _CHUNK)],
                            out_hbm.at[c, pl.ds(off, _CHUNK)])
            return carry

        lax.fori_loop(0, slc // _CHUNK, wr, 0)

    return msg_kernel(hs, src3, dst3, ew3)


def _tc_matmul_scale(x, w, deg2):
    n, f = x.shape
    h = w.shape[1]
    blk = 1000
    deg3 = deg2.reshape(_NC, -1, 1)

    def mm_body(x_ref, w_ref, o_ref):
        o_ref[...] = jnp.dot(x_ref[...], w_ref[...],
                             preferred_element_type=jnp.float32)

    hmat = pl.pallas_call(
        mm_body,
        grid=(n // blk,),
        in_specs=[
            pl.BlockSpec((blk, f), lambda i: (i, 0)),
            pl.BlockSpec((f, h), lambda i: (0, 0)),
        ],
        out_specs=pl.BlockSpec((blk, h), lambda i: (i, 0)),
        out_shape=jax.ShapeDtypeStruct((n, h), jnp.float32),
    )(x, w)

    def sc_body(h_ref, deg_ref, o_ref):
        d = deg_ref[0] + deg_ref[1] + 1.0
        o_ref[...] = h_ref[...] * lax.rsqrt(d)

    return pl.pallas_call(
        sc_body,
        grid=(n // blk,),
        in_specs=[
            pl.BlockSpec((blk, h), lambda i: (i, 0)),
            pl.BlockSpec((_NC, blk, 1), lambda i: (0, i, 0)),
        ],
        out_specs=pl.BlockSpec((blk, h), lambda i: (i, 0)),
        out_shape=jax.ShapeDtypeStruct((n, h), jnp.float32),
    )(hmat, deg3)


def _tc_pool_mlp(acc, hs, deg2, batch, rho, b1, M1, mb1, M2, mb2, M3, mb3):
    n, h = hs.shape
    g = rho.shape[0]
    out_dim = M3.shape[1]
    blk = 1000
    grid = n // blk
    deg3 = deg2.reshape(_NC, -1, 1)
    bat2 = batch.reshape(n, 1).astype(jnp.int32)
    rho2 = rho.reshape(g, 1)
    b1r = b1.reshape(1, h)
    mb1r = mb1.reshape(1, h)
    mb2r = mb2.reshape(1, h)
    mb3r = mb3.reshape(1, out_dim)
    m1a = M1[:h]
    m1b = M1[h:2 * h]
    m1c = M1[2 * h:]

    def body(acc_ref, hs_ref, deg_ref, bat_ref, rho_ref, b1_ref,
             m1a_ref, m1b_ref, m1c_ref, mb1_ref, m2_ref, mb2_ref,
             m3_ref, mb3_ref, o_ref, hsum_s, hmax_s, cnt_s):
        i = pl.program_id(0)

        @pl.when(i == 0)
        def _():
            hsum_s[...] = jnp.zeros_like(hsum_s)
            hmax_s[...] = jnp.zeros_like(hmax_s)
            cnt_s[...] = jnp.zeros_like(cnt_s)

        d = deg_ref[0] + deg_ref[1] + 1.0
        dinv = lax.rsqrt(d)
        hact = jnp.maximum(
            (acc_ref[0] + acc_ref[1] + hs_ref[...]) * dinv + b1_ref[...], 0.0)
        bat = bat_ref[...]
        gid = lax.broadcasted_iota(jnp.int32, (blk, g), 1)
        mask = (bat == gid).astype(jnp.float32)
        dn = (((0,), (0,)), ((), ()))
        hsum_s[...] += lax.dot_general(mask, hact, dn,
                                       preferred_element_type=jnp.float32)
        cnt_s[...] += lax.dot_general(mask, jnp.ones_like(hact), dn,
                                      preferred_element_type=jnp.float32)

        def gbody(gg, carry):
            m = (bat == gg).astype(jnp.float32)
            colmax = jnp.max(hact * m, axis=0, keepdims=True)
            hmax_s[pl.ds(gg, 1), :] = jnp.maximum(hmax_s[pl.ds(gg, 1), :], colmax)
            return carry

        g_lo = bat[0, 0]
        g_hi = bat[blk - 1, 0]
        lax.fori_loop(g_lo, g_hi + 1, gbody, 0)

        @pl.when(i == grid - 1)
        def _():
            hmean = hsum_s[...] / jnp.maximum(cnt_s[...], 1.0)
            hg = (jnp.dot(hmax_s[...], m1a_ref[...],
                          preferred_element_type=jnp.float32)
                  + jnp.dot(hmean, m1b_ref[...],
                            preferred_element_type=jnp.float32)
                  + rho_ref[...] * m1c_ref[...] + mb1_ref[...])
            z = jnp.maximum(hg, 0.0)
            z = jnp.maximum(
                jnp.dot(z, m2_ref[...], preferred_element_type=jnp.float32)
                + mb2_ref[...], 0.0)
            o_ref[...] = (jnp.dot(z, m3_ref[...],
                                  preferred_element_type=jnp.float32)
                          + mb3_ref[...])

    full = lambda shape: pl.BlockSpec(shape, lambda i: tuple(0 for _ in shape))
    return pl.pallas_call(
        body,
        grid=(grid,),
        in_specs=[
            pl.BlockSpec((_NC, blk, h), lambda i: (0, i, 0)),
            pl.BlockSpec((blk, h), lambda i: (i, 0)),
            pl.BlockSpec((_NC, blk, 1), lambda i: (0, i, 0)),
            pl.BlockSpec((blk, 1), lambda i: (i, 0)),
            full((g, 1)),
            full((1, h)),
            full((h, h)),
            full((h, h)),
            full((1, h)),
            full((1, h)),
            full((h, h)),
            full((1, h)),
            full((h, out_dim)),
            full((1, out_dim)),
        ],
        out_specs=pl.BlockSpec((g, out_dim), lambda i: (0, 0)),
        out_shape=jax.ShapeDtypeStruct((g, out_dim), jnp.float32),
        scratch_shapes=[
            pltpu.VMEM((g, h), jnp.float32),
            pltpu.VMEM((g, h), jnp.float32),
            pltpu.VMEM((g, h), jnp.float32),
        ],
    )(acc, hs, deg3, bat2, rho2, b1r, m1a, m1b, m1c, mb1r, M2, mb2r, M3, mb3r)


def kernel(x, edge_index, edge_attr, batch, rho, W1, b1, M1, mb1, M2, mb2, M3, mb3):
    n, _ = x.shape
    e = edge_index.shape[1]

    n_pad = -(-n // 2048) * 2048

    grp = _NW * _CHUNK
    e_pad = -(-e // grp) * grp
    pad = e_pad - e

    src = edge_index[0].astype(jnp.int32)
    dst = edge_index[1].astype(jnp.int32)
    ew = edge_attr.reshape(-1).astype(jnp.float32)
    if pad:
        fill = (jnp.arange(pad, dtype=jnp.int32) * 1009) % n
        src = jnp.concatenate([src, fill])
        dst = jnp.concatenate([dst, fill])
        ew = jnp.concatenate([ew, jnp.zeros((pad,), jnp.float32)])

    ch = e_pad // grp
    src3 = src.reshape(_NW, ch, _CHUNK)
    dst3 = dst.reshape(_NW, ch, _CHUNK)
    ew3 = ew.reshape(_NW, ch, _CHUNK)

    deg2 = _sc_degree(dst3, ew3, n_pad)
    hs = _tc_matmul_scale(x, W1, deg2)
    acc = _sc_messages(hs, src3, dst3, ew3, n_pad)
    return _tc_pool_mlp(acc, hs, deg2, batch, rho,
                        b1, M1, mb1, M2, mb2, M3, mb3)
